# Initial kernel scaffold; baseline (speedup 1.0000x reference)
import jax
import jax.numpy as jnp
from jax.experimental import pallas as pl


def kernel(real_labels):
    raise NotImplementedError("write your pallas kernel here")


# speedup vs baseline: 11.5513x; 11.5513x over previous
import functools

import jax
import jax.numpy as jnp
import numpy as np
from jax.experimental import pallas as pl

_STRIDES = [8, 16, 32]
_IMAGE_SIZE = 640
_FS = [_IMAGE_SIZE // s for s in _STRIDES]
_ANCHORS = [
    np.array([[12., 16.], [19., 36.], [40., 28.]], dtype=np.float32) / 8.0,
    np.array([[36., 75.], [76., 55.], [72., 146.]], dtype=np.float32) / 16.0,
    np.array([[142., 110.], [192., 243.], [459., 401.]], dtype=np.float32) / 32.0,
]
_K = 90
_HALF_MAX = 65504.0
_B, _N, _A = 128, 150, 3


def _head_kernel(cls_ref, x_ref, y_ref, w_ref, h_ref, out_ref, *, fs, aw, ah):
    f32 = jnp.float32
    i32 = jnp.int32
    cls2 = cls_ref[:, :]
    x2 = x_ref[:, :] * fs
    y2 = y_ref[:, :] * fs
    w2 = w_ref[:, :] * fs
    h2 = h_ref[:, :] * fs

    ni = jax.lax.broadcasted_iota(i32, (_N, _B), 0)
    bi = jax.lax.broadcasted_iota(i32, (_N, _B), 1)

    w0 = w2[:, 0:1]
    h0 = h2[:, 0:1]

    v = [None] * _A
    mult = [None] * _A
    nth = [[None] * 4 for _ in range(_A)]
    fidx = [None] * _A
    afv = [None] * _A

    for a in range(_A):
        rw = w0 / float(aw[a])
        rh = h0 / float(ah[a])
        worse = jnp.maximum(jnp.maximum(rw, 1.0 / rw),
                            jnp.maximum(rh, 1.0 / rh))
        worse = jnp.where(worse != 0.0, worse, _HALF_MAX)
        maskT = worse < 4.0

        xm = jnp.where(maskT, x2, 0.0)
        ym = jnp.where(maskT, y2, 0.0)
        b1 = (xm % 1.0 < 0.5) & (xm > 1.0)
        b2 = (ym % 1.0 < 0.5) & (ym > 1.0)
        invx = jnp.where(xm != 0.0, fs - xm, 0.0)
        invy = jnp.where(ym != 0.0, fs - ym, 0.0)
        b3 = (invx % 1.0 < 0.5) & (invx > 1.0)
        b4 = (invy % 1.0 < 0.5) & (invy > 1.0)

        v_raw = ((((cls2 + x2) + y2) + w2) + h2) + float(a + 1)
        v[a] = jnp.where(maskT, v_raw, 0.0)

        b1i = b1.astype(i32)
        b2i = b2.astype(i32)
        b3i = b3.astype(i32)
        b4i = b4.astype(i32)
        mult[a] = jnp.where(maskT, 1 + b1i + b2i + b3i + b4i, 0)

        s1 = b1i
        s2 = s1 + b2i
        s3 = s2 + b3i
        s4 = s3 + b4i
        for k in range(1, 5):
            r = (1 * b1i * (s1 == k) + 2 * b2i * (s2 == k)
                 + 3 * b3i * (s3 == k) + 4 * b4i * (s4 == k))
            nth[a][k - 1] = jnp.where(s4 < k, 7, r)

        fidx[a] = (bi * _N + ni) * _A + a
        afv[a] = float(a)

    rows8 = jax.lax.broadcasted_iota(i32, (8, 128), 0)
    lanes8 = jax.lax.broadcasted_iota(i32, (8, 128), 1)
    BIG = jnp.int32(2 ** 30)

    def body(p, carry):
        claimed, out_acc = carry
        nc = []
        rem = []
        for a in range(_A):
            ca = claimed[a]
            nc.append(jnp.where(ca == 0, 0,
                      jnp.where(ca == 1, nth[a][0],
                      jnp.where(ca == 2, nth[a][1],
                      jnp.where(ca == 3, nth[a][2], nth[a][3])))))
            rem.append(ca < mult[a])
        v_eff = [jnp.where(rem[a], v[a], -1.0) for a in range(_A)]
        m = jnp.maximum(jnp.maximum(jnp.max(v_eff[0]), jnp.max(v_eff[1])),
                        jnp.max(v_eff[2]))
        valid = m > 0.0
        t1 = [rem[a] & (v_eff[a] == m) for a in range(_A)]
        cmin = jnp.minimum(
            jnp.minimum(jnp.min(jnp.where(t1[0], nc[0], 99)),
                        jnp.min(jnp.where(t1[1], nc[1], 99))),
            jnp.min(jnp.where(t1[2], nc[2], 99)))
        t2 = [t1[a] & (nc[a] == cmin) for a in range(_A)]
        widx = jnp.minimum(
            jnp.minimum(jnp.min(jnp.where(t2[0], fidx[0], BIG)),
                        jnp.min(jnp.where(t2[1], fidx[1], BIG))),
            jnp.min(jnp.where(t2[2], fidx[2], BIG)))
        wm = [t2[a] & (fidx[a] == widx) for a in range(_A)]
        wmf = [wm[a].astype(f32) for a in range(_A)]
        wsum = wmf[0] + wmf[1] + wmf[2]

        g_cls = jnp.sum(wsum * cls2)
        g_x = jnp.sum(wsum * x2)
        g_y = jnp.sum(wsum * y2)
        g_w = jnp.sum(wsum * w2)
        g_h = jnp.sum(wsum * h2)
        g_a = jnp.sum(wmf[1]) + 2.0 * jnp.sum(wmf[2])

        offx = 0.5 * ((cmin == 1).astype(f32) - (cmin == 3).astype(f32))
        offy = 0.5 * ((cmin == 2).astype(f32) - (cmin == 4).astype(f32))

        lxi_x = jnp.where(g_x != 0.0, (g_x - offx).astype(i32), 0)
        lxi_y = jnp.where(g_y != 0.0, (g_y - offy).astype(i32), 0)
        x_ind = jnp.clip(lxi_x, 0, int(fs) - 1).astype(f32)
        y_ind = jnp.clip(lxi_y, 0, int(fs) - 1).astype(f32)
        tbx = g_x - lxi_x.astype(f32)
        tby = g_y - lxi_y.astype(f32)

        vf = valid.astype(f32)
        col = (g_a * (rows8 == 0) + y_ind * (rows8 == 1) + x_ind * (rows8 == 2)
               + g_cls * (rows8 == 3) + tbx * (rows8 == 4) + tby * (rows8 == 5)
               + g_w * (rows8 == 6) + g_h * (rows8 == 7)) * vf
        out_acc = jnp.where(lanes8 == p, col, out_acc)
        claimed = tuple(claimed[a] + jnp.where(valid, wm[a].astype(i32), 0)
                        for a in range(_A))
        return claimed, out_acc

    claimed0 = tuple(jnp.zeros((_N, _B), i32) for _ in range(_A))
    acc0 = jnp.zeros((8, 128), f32)
    _, out_acc = jax.lax.fori_loop(0, _K, body, (claimed0, acc0))
    out_ref[:, :] = out_acc


def _run_head(parts, h):
    fs = float(_FS[h])
    fn = functools.partial(_head_kernel, fs=fs,
                           aw=_ANCHORS[h][:, 0], ah=_ANCHORS[h][:, 1])
    out = pl.pallas_call(
        fn,
        out_shape=jax.ShapeDtypeStruct((8, 128), jnp.float32),
    )(*parts)
    anchor = out[0, :_K].astype(jnp.int32)
    y_ind = out[1, :_K].astype(jnp.int32)
    x_ind = out[2, :_K].astype(jnp.int32)
    t_boxes = out[3:8, :_K].T
    return anchor, y_ind, x_ind, t_boxes


@jax.jit
def kernel(real_labels):
    lt = real_labels.transpose(2, 1, 0)
    parts = tuple(lt[i] for i in range(5))
    out = ()
    for h in range(3):
        out = out + _run_head(parts, h)
    return out


# speedup vs baseline: 13.6185x; 1.1790x over previous
import functools

import jax
import jax.numpy as jnp
import numpy as np
from jax.experimental import pallas as pl

_STRIDES = [8, 16, 32]
_IMAGE_SIZE = 640
_FS = [_IMAGE_SIZE // s for s in _STRIDES]
_ANCHORS = [
    np.array([[12., 16.], [19., 36.], [40., 28.]], dtype=np.float32) / 8.0,
    np.array([[36., 75.], [76., 55.], [72., 146.]], dtype=np.float32) / 16.0,
    np.array([[142., 110.], [192., 243.], [459., 401.]], dtype=np.float32) / 32.0,
]
_K = 90
_HALF_MAX = 65504.0
_B, _N, _A = 128, 150, 3


def _head_kernel(cls_ref, x_ref, y_ref, w_ref, h_ref, out_ref, *, fs, aw, ah):
    f32 = jnp.float32
    i32 = jnp.int32
    cls2 = cls_ref[:, :]
    x2 = x_ref[:, :] * fs
    y2 = y_ref[:, :] * fs
    w2 = w_ref[:, :] * fs
    h2 = h_ref[:, :] * fs

    w0 = w2[:, 0:1]
    h0 = h2[:, 0:1]

    v_l, mult_l, nth_l = [], [], [[], [], [], []]
    for a in range(_A):
        rw = w0 / float(aw[a])
        rh = h0 / float(ah[a])
        worse = jnp.maximum(jnp.maximum(rw, 1.0 / rw),
                            jnp.maximum(rh, 1.0 / rh))
        worse = jnp.where(worse != 0.0, worse, _HALF_MAX)
        maskT = worse < 4.0

        xm = jnp.where(maskT, x2, 0.0)
        ym = jnp.where(maskT, y2, 0.0)
        b1 = (xm % 1.0 < 0.5) & (xm > 1.0)
        b2 = (ym % 1.0 < 0.5) & (ym > 1.0)
        invx = jnp.where(xm != 0.0, fs - xm, 0.0)
        invy = jnp.where(ym != 0.0, fs - ym, 0.0)
        b3 = (invx % 1.0 < 0.5) & (invx > 1.0)
        b4 = (invy % 1.0 < 0.5) & (invy > 1.0)

        v_raw = ((((cls2 + x2) + y2) + w2) + h2) + float(a + 1)
        v_l.append(jnp.where(maskT, v_raw, 0.0))

        b1i = b1.astype(i32)
        b2i = b2.astype(i32)
        b3i = b3.astype(i32)
        b4i = b4.astype(i32)
        mult_l.append(jnp.where(maskT, 1 + b1i + b2i + b3i + b4i, 0))

        s1 = b1i
        s2 = s1 + b2i
        s3 = s2 + b3i
        s4 = s3 + b4i
        for k in range(1, 5):
            r = (1 * b1i * (s1 == k) + 2 * b2i * (s2 == k)
                 + 3 * b3i * (s3 == k) + 4 * b4i * (s4 == k))
            nth_l[k - 1].append(jnp.where(s4 < k, 7, r))

    cat = lambda xs: jnp.concatenate(xs, axis=0)
    v = cat(v_l)
    mult = cat(mult_l)
    n1, n2, n3, n4 = (cat(nth_l[k]) for k in range(4))
    c3 = cat([cls2] * _A)
    x3 = cat([x2] * _A)
    y3 = cat([y2] * _A)
    w3 = cat([w2] * _A)
    h3 = cat([h2] * _A)

    S = (_A * _N, _B)
    ri = jax.lax.broadcasted_iota(i32, S, 0)
    bi = jax.lax.broadcasted_iota(i32, S, 1)
    ai = ri // _N
    nn = ri - ai * _N
    fidx = (bi * _N + nn) * _A + ai

    rows8 = jax.lax.broadcasted_iota(i32, (8, 128), 0)
    lanes8 = jax.lax.broadcasted_iota(i32, (8, 128), 1)
    BIG = jnp.int32(2 ** 30)

    def body(p, carry):
        claimed, out_acc = carry
        nc = jnp.where(claimed == 0, 0,
             jnp.where(claimed == 1, n1,
             jnp.where(claimed == 2, n2,
             jnp.where(claimed == 3, n3, n4))))
        rem = claimed < mult
        v_eff = jnp.where(rem, v, -1.0)
        m = jnp.max(v_eff)
        valid = m > 0.0
        t1 = rem & (v_eff == m)
        key = nc * 65536 + fidx
        kmin = jnp.min(jnp.where(t1, key, BIG))
        cmin = kmin // 65536
        widx = kmin - cmin * 65536
        wm = t1 & (key == kmin)
        wmf = wm.astype(f32)

        g_cls = jnp.sum(wmf * c3)
        g_x = jnp.sum(wmf * x3)
        g_y = jnp.sum(wmf * y3)
        g_w = jnp.sum(wmf * w3)
        g_h = jnp.sum(wmf * h3)
        g_a = (widx - (widx // _A) * _A).astype(f32)

        offx = 0.5 * ((cmin == 1).astype(f32) - (cmin == 3).astype(f32))
        offy = 0.5 * ((cmin == 2).astype(f32) - (cmin == 4).astype(f32))

        lxi_x = jnp.where(g_x != 0.0, (g_x - offx).astype(i32), 0)
        lxi_y = jnp.where(g_y != 0.0, (g_y - offy).astype(i32), 0)
        x_ind = jnp.clip(lxi_x, 0, int(fs) - 1).astype(f32)
        y_ind = jnp.clip(lxi_y, 0, int(fs) - 1).astype(f32)
        tbx = g_x - lxi_x.astype(f32)
        tby = g_y - lxi_y.astype(f32)

        vf = valid.astype(f32)
        col = (g_a * (rows8 == 0) + y_ind * (rows8 == 1) + x_ind * (rows8 == 2)
               + g_cls * (rows8 == 3) + tbx * (rows8 == 4) + tby * (rows8 == 5)
               + g_w * (rows8 == 6) + g_h * (rows8 == 7)) * vf
        out_acc = jnp.where(lanes8 == p, col, out_acc)
        claimed = claimed + jnp.where(valid & wm, 1, 0)
        return claimed, out_acc

    claimed0 = jnp.zeros(S, i32)
    acc0 = jnp.zeros((8, 128), f32)
    _, out_acc = jax.lax.fori_loop(0, _K, body, (claimed0, acc0))
    out_ref[:, :] = out_acc


def _run_head(parts, h):
    fs = float(_FS[h])
    fn = functools.partial(_head_kernel, fs=fs,
                           aw=_ANCHORS[h][:, 0], ah=_ANCHORS[h][:, 1])
    out = pl.pallas_call(
        fn,
        out_shape=jax.ShapeDtypeStruct((8, 128), jnp.float32),
    )(*parts)
    anchor = out[0, :_K].astype(jnp.int32)
    y_ind = out[1, :_K].astype(jnp.int32)
    x_ind = out[2, :_K].astype(jnp.int32)
    t_boxes = out[3:8, :_K].T
    return anchor, y_ind, x_ind, t_boxes


@jax.jit
def kernel(real_labels):
    lt = real_labels.transpose(2, 1, 0)
    parts = tuple(lt[i] for i in range(5))
    out = ()
    for h in range(3):
        out = out + _run_head(parts, h)
    return out
